# Initial kernel scaffold; baseline (speedup 1.0000x reference)
import jax
import jax.numpy as jnp
from jax.experimental import pallas as pl


def kernel(points):
    raise NotImplementedError("write your pallas kernel here")


# speedup vs baseline: 12.8813x; 12.8813x over previous
import jax
import jax.numpy as jnp
from jax.experimental import pallas as pl
from jax.experimental.pallas import tpu as pltpu


def _nms_kernel(x_ref, o_ref):
    x = x_ref[...]
    zc = jnp.zeros_like(x[:, :, :1])
    m = jnp.maximum(x, jnp.concatenate([x[:, :, 1:], zc], axis=2))
    m = jnp.maximum(m, jnp.concatenate([zc, x[:, :, :-1]], axis=2))
    zr = jnp.zeros_like(m[:, :1, :])
    lm = jnp.maximum(m, jnp.concatenate([m[:, 1:, :], zr], axis=1))
    lm = jnp.maximum(lm, jnp.concatenate([zr, m[:, :-1, :]], axis=1))
    o_ref[...] = jnp.where(x == lm, x, 0.0)


def kernel(points):
    b, c, h, w = points.shape
    flat = points.reshape(b * c, h, w)
    cb = 40
    out = pl.pallas_call(
        _nms_kernel,
        out_shape=jax.ShapeDtypeStruct(flat.shape, flat.dtype),
        grid=(flat.shape[0] // cb,),
        in_specs=[pl.BlockSpec((cb, h, w), lambda i: (i, 0, 0))],
        out_specs=pl.BlockSpec((cb, h, w), lambda i: (i, 0, 0)),
        compiler_params=pltpu.CompilerParams(
            dimension_semantics=("parallel",),
        ),
    )(flat)
    return out.reshape(b, c, h, w)
